# baseline (speedup 1.0000x reference)
import functools

import jax
import jax.numpy as jnp
from jax import lax
from jax.experimental import pallas as pl
from jax.experimental.pallas import tpu as pltpu
from jax.experimental.pallas import tpu_sc as plsc

_M = 128
_N = 32768
_LANES = 16
_NVEC = _N // _LANES
_UNROLL = 8
_NEG = float("-inf")
_BIG = 2**30


def _gather16(v, perm):
    return lax.gather(
        v, perm[:, None],
        dimension_numbers=lax.GatherDimensionNumbers(
            offset_dims=(), collapsed_slice_dims=(0,), start_index_map=(0,)),
        slice_sizes=(1,),
        mode=lax.GatherScatterMode.PROMISE_IN_BOUNDS)


def _sc_kernel_body(x_hbm, vout_hbm, iout_hbm,
                    buf0, buf1, outv, outi, sem0, sem1):
    nc = 2
    wid = lax.axis_index("s") * nc + lax.axis_index("c")
    base = wid * 4
    iota = lax.iota(jnp.int32, _LANES)

    bufs = (buf0, buf1)
    sems = (sem0, sem1)
    copies = [None, None]

    copies[0] = pltpu.async_copy(x_hbm.at[base], buf0, sem0)

    for r in range(4):
        if r + 1 < 4:
            copies[(r + 1) % 2] = pltpu.async_copy(
                x_hbm.at[base + r + 1], bufs[(r + 1) % 2], sems[(r + 1) % 2])
        copies[r % 2].wait()
        row = bufs[r % 2]

        neg = jnp.full((_LANES,), _NEG, jnp.float32)
        zero = jnp.zeros((_LANES,), jnp.int32)

        def body(c, carry):
            t1, t2, t3, i1, i2, i3, ibase = carry
            for u in range(_UNROLL):
                j = c * _UNROLL + u
                v = row[pl.ds(j * _LANES, _LANES)]
                iv = ibase + (u * _LANES)
                m1 = v > t1
                m2 = v > t2
                m3 = v > t3
                t3 = jnp.where(m2, t2, jnp.where(m3, v, t3))
                i3 = jnp.where(m2, i2, jnp.where(m3, iv, i3))
                t2 = jnp.where(m1, t1, jnp.where(m2, v, t2))
                i2 = jnp.where(m1, i1, jnp.where(m2, iv, i2))
                t1 = jnp.where(m1, v, t1)
                i1 = jnp.where(m1, iv, i1)
            ibase = ibase + (_UNROLL * _LANES)
            return t1, t2, t3, i1, i2, i3, ibase

        t1, t2, t3, i1, i2, i3, _ = lax.fori_loop(
            0, _NVEC // _UNROLL, body,
            (neg, neg, neg, zero, zero, zero, iota))

        vvec = jnp.zeros((_LANES,), jnp.float32)
        ivec = jnp.zeros((_LANES,), jnp.int32)
        for k in range(3):
            bt, bi = t1, i1
            for s in (1, 2, 4, 8):
                perm = iota ^ s
                ot = _gather16(bt, perm)
                oi = _gather16(bi, perm)
                take = (ot > bt) | ((ot == bt) & (oi < bi))
                bt = jnp.where(take, ot, bt)
                bi = jnp.where(take, oi, bi)
            vvec = jnp.where(iota == k, bt, vvec)
            ivec = jnp.where(iota == k, bi, ivec)
            if k < 2:
                hit = i1 == bi
                t1 = jnp.where(hit, t2, t1)
                i1 = jnp.where(hit, i2, i1)
                t2 = jnp.where(hit, t3, t2)
                i2 = jnp.where(hit, i3, i2)
                t3 = jnp.where(hit, jnp.float32(_NEG), t3)
        outv[r] = vvec
        outi[r] = ivec

    pltpu.sync_copy(outv, vout_hbm.at[pl.ds(base, 4)])
    pltpu.sync_copy(outi, iout_hbm.at[pl.ds(base, 4)])


def kernel(x):
    mesh = plsc.VectorSubcoreMesh(core_axis_name="c", subcore_axis_name="s")
    k = functools.partial(
        pl.kernel,
        mesh=mesh,
        out_type=[
            jax.ShapeDtypeStruct((_M, _LANES), jnp.float32),
            jax.ShapeDtypeStruct((_M, _LANES), jnp.int32),
        ],
        scratch_types=[
            pltpu.VMEM((_N,), jnp.float32),
            pltpu.VMEM((_N,), jnp.float32),
            pltpu.VMEM((4, _LANES), jnp.float32),
            pltpu.VMEM((4, _LANES), jnp.int32),
            pltpu.SemaphoreType.DMA,
            pltpu.SemaphoreType.DMA,
        ],
    )(_sc_kernel_body)
    v, i = k(x)
    return (v[:, :3], i[:, :3])
